# baseline (speedup 1.0000x reference)
import functools

import jax
import jax.numpy as jnp
from jax import lax
from jax.experimental import pallas as pl
from jax.experimental.pallas import tpu as pltpu
from jax.experimental.pallas import tpu_sc as plsc

_VOCAB = 1000
_EMB = 32
_T = 8

_NC = 2
_NS = 16
_NW = _NC * _NS

_GATHER_ROWS = 128
_GROUP = 8


_L = 16
_CHUNK_ROWS = 1024


def _sc_gather(tok_flat, idx2d):
  n_vecs, l = idx2d.shape
  n_total = n_vecs * l
  vecs_per_w = n_vecs // _NW
  rows_per_w = vecs_per_w * l
  vecs_per_chunk = _CHUNK_ROWS // l
  chunks_per_w = rows_per_w // _CHUNK_ROWS
  mesh = plsc.VectorSubcoreMesh(core_axis_name="c", subcore_axis_name="s")

  @functools.partial(
      pl.kernel,
      mesh=mesh,
      out_type=jax.ShapeDtypeStruct((n_total * _EMB,), jnp.float32),
      compiler_params=pltpu.CompilerParams(use_tc_tiling_on_sc=False,
                                           needs_layout_passes=False),
      scratch_types=[
          pltpu.VMEM((_VOCAB * _EMB,), jnp.float32),
          pltpu.VMEM((vecs_per_w, l), jnp.int32),
          pltpu.VMEM((_CHUNK_ROWS * _EMB,), jnp.float32),
          pltpu.VMEM((_CHUNK_ROWS * _EMB,), jnp.float32),
          pltpu.SemaphoreType.DMA,
          pltpu.SemaphoreType.DMA,
      ],
  )
  def k(table_hbm, idx_hbm, out_hbm, table_v, idx_v, buf_a, buf_b, wsem_a,
        wsem_b):
    wid = lax.axis_index("s") * _NC + lax.axis_index("c")
    base_vec = wid * vecs_per_w
    pltpu.sync_copy(table_hbm, table_v)
    pltpu.sync_copy(idx_hbm.at[pl.ds(base_vec, vecs_per_w)], idx_v)

    lane = lax.iota(jnp.int32, _L)
    soff0 = lane * _EMB

    def chunk_rows(buf, c):
      def vec_body(j, carry):
        base = idx_v[c * vecs_per_chunk + j] * _EMB
        soff = soff0 + j * (_L * _EMB)
        for col in range(_EMB):
          vals = plsc.load_gather(table_v, [base + col])
          plsc.store_scatter(buf, [soff + col], vals)
        return carry

      lax.fori_loop(0, vecs_per_chunk, vec_body, 0)

    bufs = [buf_a, buf_b]
    sems = [wsem_a, wsem_b]
    writes = [None, None]
    for c in range(chunks_per_w):
      buf = bufs[c % 2]
      if writes[c % 2] is not None:
        writes[c % 2].wait()
      chunk_rows(buf, c)
      elem0 = (wid * rows_per_w + c * _CHUNK_ROWS) * _EMB
      writes[c % 2] = pltpu.async_copy(
          buf, out_hbm.at[pl.ds(elem0, _CHUNK_ROWS * _EMB)], sems[c % 2])
    for wr in writes:
      if wr is not None:
        wr.wait()

  return k(tok_flat, idx2d).reshape(n_total, _EMB)


def _tc_head(x, pos_table, w, b2, tgt2, block_rows):
  n = x.shape[0]
  grid = n // block_rows

  def body(x_ref, pos_ref, w_ref, b_ref, t_ref, out_ref, loss_ref):
    i = pl.program_id(0)
    wmat = w_ref[...]
    logits = jnp.dot(x_ref[...], wmat,
                     preferred_element_type=jnp.float32)
    posw = jnp.dot(pos_ref[...], wmat,
                   preferred_element_type=jnp.float32)
    pb = posw + b_ref[...]
    pb_full = jnp.broadcast_to(
        pb[None], (block_rows // _T, _T, _VOCAB)).reshape(block_rows, _VOCAB)
    logits = logits + pb_full
    out_ref[...] = logits

    m = jnp.max(logits, axis=1, keepdims=True)
    s = jnp.sum(jnp.exp(logits - m), axis=1, keepdims=True)
    lse = m + jnp.log(s)
    col = lax.broadcasted_iota(jnp.int32, (block_rows, _VOCAB), 1)
    tl = jnp.sum(jnp.where(col == t_ref[...], logits, 0.0),
                 axis=1, keepdims=True)
    part = jnp.sum(lse - tl).reshape(1, 1)

    @pl.when(i == 0)
    def _():
      loss_ref[...] = jnp.zeros((1, 1), jnp.float32)

    loss_ref[...] += part

  return pl.pallas_call(
      body,
      grid=(grid,),
      in_specs=[
          pl.BlockSpec((block_rows, _EMB), lambda i: (i, 0)),
          pl.BlockSpec((_T, _EMB), lambda i: (0, 0)),
          pl.BlockSpec((_EMB, _VOCAB), lambda i: (0, 0)),
          pl.BlockSpec((1, _VOCAB), lambda i: (0, 0)),
          pl.BlockSpec((block_rows, 1), lambda i: (i, 0)),
      ],
      out_specs=[
          pl.BlockSpec((block_rows, _VOCAB), lambda i: (i, 0)),
          pl.BlockSpec((1, 1), lambda i: (0, 0)),
      ],
      out_shape=[
          jax.ShapeDtypeStruct((n, _VOCAB), jnp.float32),
          jax.ShapeDtypeStruct((1, 1), jnp.float32),
      ],
      compiler_params=pltpu.CompilerParams(
          vmem_limit_bytes=100 * 1024 * 1024),
  )(x, pos_table, w, b2, tgt2)


def kernel(inputs, targets, tok_table, pos_table, W, b):
  bd, td = inputs.shape
  n = bd * td
  idx2d = inputs.reshape(n // _L, _L).astype(jnp.int32)
  x = _sc_gather(tok_table.reshape(-1), idx2d)
  tgt2 = targets.reshape(n, 1).astype(jnp.int32)
  logits, loss_sum = _tc_head(x, pos_table, W, b.reshape(1, _VOCAB), tgt2,
                              block_rows=4096)
  return logits, loss_sum[0, 0] / n


# speedup vs baseline: 1.1262x; 1.1262x over previous
import functools

import jax
import jax.numpy as jnp
from jax import lax
from jax.experimental import pallas as pl
from jax.experimental.pallas import tpu as pltpu
from jax.experimental.pallas import tpu_sc as plsc

_VOCAB = 1000
_EMB = 32
_T = 8

_NC = 2
_NS = 16
_NW = _NC * _NS

_GATHER_ROWS = 256
_GROUP = 4


def _sc_gather(tok_table, idx2d):
  n_streams, g = idx2d.shape
  n_total = n_streams * g
  streams_per_w = n_streams // _NW
  groups_per_w = streams_per_w // _GROUP
  rows_per_group = _GROUP * g
  mesh = plsc.VectorSubcoreMesh(core_axis_name="c", subcore_axis_name="s")

  @functools.partial(
      pl.kernel,
      mesh=mesh,
      out_type=jax.ShapeDtypeStruct((n_total, _EMB), jnp.float32),
      compiler_params=pltpu.CompilerParams(use_tc_tiling_on_sc=False),
      scratch_types=[
          pltpu.VMEM((streams_per_w, g), jnp.int32),
          pltpu.VMEM((rows_per_group, _EMB), jnp.float32),
          pltpu.VMEM((rows_per_group, _EMB), jnp.float32),
          pltpu.SemaphoreType.DMA,
          pltpu.SemaphoreType.DMA,
          pltpu.SemaphoreType.DMA,
      ],
  )
  def k(table_hbm, idx_hbm, out_hbm, idx_v, buf_a, buf_b, gsem_a, gsem_b,
        wsem):
    wid = lax.axis_index("s") * _NC + lax.axis_index("c")
    base_stream = wid * streams_per_w
    pltpu.sync_copy(idx_hbm.at[pl.ds(base_stream, streams_per_w)], idx_v)

    def pair_body(j, carry):
      ga = [
          pltpu.async_copy(
              table_hbm.at[idx_v.at[(2 * j) * _GROUP + t]],
              buf_a.at[pl.ds(t * g, g)], gsem_a)
          for t in range(_GROUP)
      ]
      gb = [
          pltpu.async_copy(
              table_hbm.at[idx_v.at[(2 * j + 1) * _GROUP + t]],
              buf_b.at[pl.ds(t * g, g)], gsem_b)
          for t in range(_GROUP)
      ]
      for cp in ga:
        cp.wait()
      row_a = (base_stream + (2 * j) * _GROUP) * g
      wa = pltpu.async_copy(buf_a, out_hbm.at[pl.ds(row_a, rows_per_group)],
                            wsem)
      for cp in gb:
        cp.wait()
      row_b = (base_stream + (2 * j + 1) * _GROUP) * g
      wb = pltpu.async_copy(buf_b, out_hbm.at[pl.ds(row_b, rows_per_group)],
                            wsem)
      wa.wait()
      wb.wait()
      return carry

    lax.fori_loop(0, groups_per_w // 2, pair_body, 0)

  return k(tok_table, idx2d)


def _tc_head(x, pos_table, w, b2, tgt2, block_rows):
  n = x.shape[0]
  grid = n // block_rows

  def body(x_ref, pos_ref, w_ref, b_ref, t_ref, out_ref, loss_ref):
    i = pl.program_id(0)
    wmat = w_ref[...]
    logits = jnp.dot(x_ref[...], wmat,
                     preferred_element_type=jnp.float32)
    posw = jnp.dot(pos_ref[...], wmat,
                   preferred_element_type=jnp.float32)
    pb = posw + b_ref[...]
    pb_full = jnp.broadcast_to(
        pb[None], (block_rows // _T, _T, _VOCAB)).reshape(block_rows, _VOCAB)
    logits = logits + pb_full
    out_ref[...] = logits

    m = jnp.max(logits, axis=1, keepdims=True)
    s = jnp.sum(jnp.exp(logits - m), axis=1, keepdims=True)
    lse = m + jnp.log(s)
    col = lax.broadcasted_iota(jnp.int32, (block_rows, _VOCAB), 1)
    tl = jnp.sum(jnp.where(col == t_ref[...], logits, 0.0),
                 axis=1, keepdims=True)
    part = jnp.sum(lse - tl).reshape(1, 1)

    @pl.when(i == 0)
    def _():
      loss_ref[...] = jnp.zeros((1, 1), jnp.float32)

    loss_ref[...] += part

  return pl.pallas_call(
      body,
      grid=(grid,),
      in_specs=[
          pl.BlockSpec((block_rows, _EMB), lambda i: (i, 0)),
          pl.BlockSpec((_T, _EMB), lambda i: (0, 0)),
          pl.BlockSpec((_EMB, _VOCAB), lambda i: (0, 0)),
          pl.BlockSpec((1, _VOCAB), lambda i: (0, 0)),
          pl.BlockSpec((block_rows, 1), lambda i: (i, 0)),
      ],
      out_specs=[
          pl.BlockSpec((block_rows, _VOCAB), lambda i: (i, 0)),
          pl.BlockSpec((1, 1), lambda i: (0, 0)),
      ],
      out_shape=[
          jax.ShapeDtypeStruct((n, _VOCAB), jnp.float32),
          jax.ShapeDtypeStruct((1, 1), jnp.float32),
      ],
      compiler_params=pltpu.CompilerParams(
          vmem_limit_bytes=100 * 1024 * 1024),
  )(x, pos_table, w, b2, tgt2)


def kernel(inputs, targets, tok_table, pos_table, W, b):
  bd, td = inputs.shape
  n = bd * td
  idx2d = inputs.reshape(n // _GATHER_ROWS, _GATHER_ROWS).astype(jnp.int32)
  x = _sc_gather(tok_table, idx2d)
  tgt2 = targets.reshape(n, 1).astype(jnp.int32)
  logits, loss_sum = _tc_head(x, pos_table, W, b.reshape(1, _VOCAB), tgt2,
                              block_rows=4096)
  return logits, loss_sum[0, 0] / n
